# Initial kernel scaffold; baseline (speedup 1.0000x reference)
import jax
import jax.numpy as jnp
from jax.experimental import pallas as pl


def kernel(x, triple_index, triple_attr, W1, b1, W2, b2, W3, b3, Wout):
    raise NotImplementedError("write your pallas kernel here")


# speedup vs baseline: 2.1053x; 2.1053x over previous
import functools

import jax
import jax.numpy as jnp
import numpy as np
from jax import lax
from jax.experimental import pallas as pl
from jax.experimental.pallas import tpu as pltpu
from jax.experimental.pallas import tpu_sc as plsc

N_NODES = 10000
N_ANGLES = 320000
SCALAR_DIM = 128
HIDDEN = 64

NC = 2
NS = 16
NW = NC * NS
IDXW = 128
RPW = 80
PAD_A = NW * RPW * IDXW
ROWS = PAD_A // IDXW
NT = 10240
STRIPE = NT // NS
CH = 512
STEPS = RPW // 4
HIDHALF = HIDDEN // 2

f32 = jnp.float32


def _k1_body(x_ref, wa_ref, wb_ref, gi_ref, gk_ref):
    xv = x_ref[...]
    gi_ref[...] = jnp.dot(xv, wa_ref[...], preferred_element_type=f32)
    gk_ref[...] = jnp.dot(xv, wb_ref[...], preferred_element_type=f32)


def _k2_body(ea_ref, eb_ref, at_ref, w1c_ref, b1_ref, w2_ref, b2_ref,
             ha_ref, hb_ref):
    e = ea_ref[...] + eb_ref[...]
    a = at_ref[...]
    h1 = (e + a[:, 0:1] * w1c_ref[0:1, :] + a[:, 1:2] * w1c_ref[1:2, :]
          + b1_ref[...])
    h1 = h1 * jax.nn.sigmoid(h1)
    h2 = jnp.dot(h1, w2_ref[...], preferred_element_type=f32) + b2_ref[...]
    h2 = h2 * jax.nn.sigmoid(h2)
    ha_ref[...] = h2[:, 0:HIDHALF]
    hb_ref[...] = h2[:, HIDHALF:HIDDEN]


def _k3_body(s0a_ref, s1a_ref, s0b_ref, s1b_ref, c_ref,
             w3a_ref, w3b_ref, b3_ref, wo_ref, o_ref):
    cnt = jnp.transpose(jnp.sum(c_ref[...], axis=0, keepdims=True))
    recip = 1.0 / jnp.maximum(cnt, 1.0)
    mean_a = (s0a_ref[...] + s1a_ref[...]) * recip
    mean_b = (s0b_ref[...] + s1b_ref[...]) * recip
    pre = (jnp.dot(mean_a, w3a_ref[...], preferred_element_type=f32)
           + jnp.dot(mean_b, w3b_ref[...], preferred_element_type=f32)
           + b3_ref[...])
    pre = jnp.where(cnt > 0.5, pre, 0.0)
    o_ref[...] = jnp.dot(pre, wo_ref[...],
                         preferred_element_type=f32) * (1.0 / np.sqrt(128.0))


_MESH = plsc.VectorSubcoreMesh(core_axis_name="c", subcore_axis_name="s")


@functools.partial(
    pl.kernel,
    out_type=(jax.ShapeDtypeStruct((PAD_A, HIDDEN), f32),
              jax.ShapeDtypeStruct((PAD_A, HIDDEN), f32)),
    mesh=_MESH,
    scratch_types=[
        pltpu.VMEM((RPW, IDXW), jnp.int32),
        pltpu.VMEM((RPW, IDXW), jnp.int32),
        pltpu.VMEM((CH, HIDDEN), f32),
        pltpu.VMEM((CH, HIDDEN), f32),
        pltpu.SemaphoreType.DMA,
        pltpu.SemaphoreType.DMA,
    ],
    compiler_params=pltpu.CompilerParams(use_tc_tiling_on_sc=False),
)
def _s1_gather(gi_hbm, gk_hbm, i2_hbm, k2_hbm, ea_hbm, eb_hbm,
               ii_v, kk_v, ba, bb, sa, sb):
    wid = lax.axis_index("s") * NC + lax.axis_index("c")
    r0 = wid * RPW
    pltpu.sync_copy(i2_hbm.at[pl.ds(r0, RPW)], ii_v)
    pltpu.sync_copy(k2_hbm.at[pl.ds(r0, RPW)], kk_v)

    def step(g, carry):
        handles = []
        for s in range(4):
            r = g * 4 + s
            handles.append(pltpu.async_copy(
                gi_hbm.at[ii_v.at[r]], ba.at[pl.ds(s * IDXW, IDXW)], sa))
            handles.append(pltpu.async_copy(
                gk_hbm.at[kk_v.at[r]], bb.at[pl.ds(s * IDXW, IDXW)], sb))
        for h in handles:
            h.wait()
        base = (r0 + g * 4) * IDXW
        pltpu.sync_copy(ba, ea_hbm.at[pl.ds(base, CH)])
        pltpu.sync_copy(bb, eb_hbm.at[pl.ds(base, CH)])
        return carry

    lax.fori_loop(0, STEPS, step, 0)


@functools.partial(
    pl.kernel,
    out_type=(pltpu.HBM((NT, HIDHALF), f32),
              pltpu.HBM((NT, HIDHALF), f32),
              pltpu.HBM((NT, HIDHALF), f32),
              pltpu.HBM((NT, HIDHALF), f32),
              pltpu.HBM((NW, NT), f32)),
    mesh=_MESH,
    scratch_types=[
        pltpu.VMEM((RPW, IDXW), jnp.int32),
        pltpu.VMEM((CH, HIDHALF), f32),
        pltpu.VMEM((NT,), f32),
        pltpu.VMEM((STRIPE, HIDHALF), f32),
        pltpu.VMEM_SHARED((NT, HIDHALF), f32),
    ],
    compiler_params=pltpu.CompilerParams(use_tc_tiling_on_sc=False,
                                         needs_layout_passes=False),
)
def _s2_scatter(j2_hbm, ha_hbm, hb_hbm, s0a_hbm, s1a_hbm, s0b_hbm, s1b_hbm,
                cnt_hbm, jv, hb, cnt_t, zs, ssum):
    cid = lax.axis_index("c")
    sid = lax.axis_index("s")
    wid = sid * NC + cid
    r0 = wid * RPW
    nr0 = sid * STRIPE
    pltpu.sync_copy(j2_hbm.at[pl.ds(r0, RPW)], jv)

    def zfill(t, carry):
        for q in range(HIDHALF // 16):
            zs[t, pl.ds(q * 16, 16)] = jnp.zeros((16,), f32)
        return carry

    def cfill(t, carry):
        cnt_t[pl.ds(t * 16, 16)] = jnp.zeros((16,), f32)
        return carry

    lax.fori_loop(0, STRIPE, zfill, 0)
    lax.fori_loop(0, NT // 16, cfill, 0)
    pltpu.sync_copy(zs, ssum.at[pl.ds(nr0, STRIPE)])
    plsc.subcore_barrier()

    ones16 = jnp.ones((16,), f32)

    def step_a(g, carry):
        base = (r0 + g * 4) * IDXW
        pltpu.sync_copy(ha_hbm.at[pl.ds(base, CH)], hb)
        for s in range(4):
            r = g * 4 + s
            pltpu.sync_copy(hb.at[pl.ds(s * IDXW, IDXW)],
                            ssum.at[jv.at[r]], add=True)
            for q in range(IDXW // 16):
                idx = jv[r, pl.ds(q * 16, 16)]
                plsc.addupdate_scatter(cnt_t, [idx], ones16)
        return carry

    lax.fori_loop(0, STEPS, step_a, 0)
    plsc.subcore_barrier()

    pltpu.sync_copy(cnt_t, cnt_hbm.at[wid])
    pltpu.sync_copy(ssum.at[pl.ds(nr0, STRIPE)], zs)

    @pl.when(cid == 0)
    def _():
        pltpu.sync_copy(zs, s0a_hbm.at[pl.ds(nr0, STRIPE)])

    @pl.when(cid == 1)
    def _():
        pltpu.sync_copy(zs, s1a_hbm.at[pl.ds(nr0, STRIPE)])

    def zfill2(t, carry):
        for q in range(HIDHALF // 16):
            zs[t, pl.ds(q * 16, 16)] = jnp.zeros((16,), f32)
        return carry

    lax.fori_loop(0, STRIPE, zfill2, 0)
    pltpu.sync_copy(zs, ssum.at[pl.ds(nr0, STRIPE)])
    plsc.subcore_barrier()

    def step_b(g, carry):
        base = (r0 + g * 4) * IDXW
        pltpu.sync_copy(hb_hbm.at[pl.ds(base, CH)], hb)
        for s in range(4):
            r = g * 4 + s
            pltpu.sync_copy(hb.at[pl.ds(s * IDXW, IDXW)],
                            ssum.at[jv.at[r]], add=True)
        return carry

    lax.fori_loop(0, STEPS, step_b, 0)
    plsc.subcore_barrier()

    pltpu.sync_copy(ssum.at[pl.ds(nr0, STRIPE)], zs)

    @pl.when(cid == 0)
    def _():
        pltpu.sync_copy(zs, s0b_hbm.at[pl.ds(nr0, STRIPE)])

    @pl.when(cid == 1)
    def _():
        pltpu.sync_copy(zs, s1b_hbm.at[pl.ds(nr0, STRIPE)])


def kernel(x, triple_index, triple_attr, W1, b1, W2, b2, W3, b3, Wout):
    i_idx = triple_index[0]
    j_idx = triple_index[1]
    k_idx = triple_index[2]
    pad = PAD_A - N_ANGLES
    i2 = jnp.concatenate([i_idx, jnp.zeros((pad,), jnp.int32)]).reshape(ROWS, IDXW)
    k2 = jnp.concatenate([k_idx, jnp.zeros((pad,), jnp.int32)]).reshape(ROWS, IDXW)
    j2 = jnp.concatenate(
        [j_idx, jnp.full((pad,), N_NODES, jnp.int32)]).reshape(ROWS, IDXW)
    attr_p = jnp.concatenate([triple_attr, jnp.zeros((pad, 2), f32)])

    W1a = W1[0:SCALAR_DIM]
    W1b = W1[SCALAR_DIM:2 * SCALAR_DIM]
    W1c = W1[2 * SCALAR_DIM:]
    b1r = b1.reshape(1, HIDDEN)
    b2r = b2.reshape(1, HIDDEN)
    b3r = b3.reshape(1, SCALAR_DIM)

    gi, gk = pl.pallas_call(
        _k1_body,
        grid=(10,),
        in_specs=[
            pl.BlockSpec((1000, SCALAR_DIM), lambda m: (m, 0)),
            pl.BlockSpec((SCALAR_DIM, HIDDEN), lambda m: (0, 0)),
            pl.BlockSpec((SCALAR_DIM, HIDDEN), lambda m: (0, 0)),
        ],
        out_specs=[
            pl.BlockSpec((1000, HIDDEN), lambda m: (m, 0)),
            pl.BlockSpec((1000, HIDDEN), lambda m: (m, 0)),
        ],
        out_shape=[
            jax.ShapeDtypeStruct((N_NODES, HIDDEN), f32),
            jax.ShapeDtypeStruct((N_NODES, HIDDEN), f32),
        ],
    )(x, W1a, W1b)

    ea, eb = _s1_gather(gi, gk, i2, k2)

    BK2 = 1024
    h2a, h2b = pl.pallas_call(
        _k2_body,
        grid=(PAD_A // BK2,),
        in_specs=[
            pl.BlockSpec((BK2, HIDDEN), lambda m: (m, 0)),
            pl.BlockSpec((BK2, HIDDEN), lambda m: (m, 0)),
            pl.BlockSpec((BK2, 2), lambda m: (m, 0)),
            pl.BlockSpec((2, HIDDEN), lambda m: (0, 0)),
            pl.BlockSpec((1, HIDDEN), lambda m: (0, 0)),
            pl.BlockSpec((HIDDEN, HIDDEN), lambda m: (0, 0)),
            pl.BlockSpec((1, HIDDEN), lambda m: (0, 0)),
        ],
        out_specs=[
            pl.BlockSpec((BK2, HIDHALF), lambda m: (m, 0)),
            pl.BlockSpec((BK2, HIDHALF), lambda m: (m, 0)),
        ],
        out_shape=[
            jax.ShapeDtypeStruct((PAD_A, HIDHALF), f32),
            jax.ShapeDtypeStruct((PAD_A, HIDHALF), f32),
        ],
    )(ea, eb, attr_p, W1c, b1r, W2, b2r)

    s0a, s1a, s0b, s1b, cpart = _s2_scatter(j2, h2a, h2b)

    BK3 = 1024
    out_p = pl.pallas_call(
        _k3_body,
        grid=(NT // BK3,),
        in_specs=[
            pl.BlockSpec((BK3, HIDHALF), lambda m: (m, 0)),
            pl.BlockSpec((BK3, HIDHALF), lambda m: (m, 0)),
            pl.BlockSpec((BK3, HIDHALF), lambda m: (m, 0)),
            pl.BlockSpec((BK3, HIDHALF), lambda m: (m, 0)),
            pl.BlockSpec((NW, BK3), lambda m: (0, m)),
            pl.BlockSpec((HIDHALF, SCALAR_DIM), lambda m: (0, 0)),
            pl.BlockSpec((HIDHALF, SCALAR_DIM), lambda m: (0, 0)),
            pl.BlockSpec((1, SCALAR_DIM), lambda m: (0, 0)),
            pl.BlockSpec((SCALAR_DIM, SCALAR_DIM), lambda m: (0, 0)),
        ],
        out_specs=pl.BlockSpec((BK3, SCALAR_DIM), lambda m: (m, 0)),
        out_shape=jax.ShapeDtypeStruct((NT, SCALAR_DIM), f32),
    )(s0a, s1a, s0b, s1b, cpart, W3[0:HIDHALF], W3[HIDHALF:HIDDEN], b3r, Wout)

    return out_p[:N_NODES]


# speedup vs baseline: 2.1245x; 1.0091x over previous
import functools

import jax
import jax.numpy as jnp
import numpy as np
from jax import lax
from jax.experimental import pallas as pl
from jax.experimental.pallas import tpu as pltpu
from jax.experimental.pallas import tpu_sc as plsc

N_NODES = 10000
N_ANGLES = 320000
SCALAR_DIM = 128
HIDDEN = 64

NC = 2
NS = 16
NW = NC * NS
IDXW = 128
RPW = 80
PAD_A = NW * RPW * IDXW
ROWS = PAD_A // IDXW
NT = 10240
STRIPE = NT // NS
CH = 512
STEPS = RPW // 4
HIDHALF = HIDDEN // 2

f32 = jnp.float32


def _k1_body(x_ref, wa_ref, wb_ref, gi_ref, gk_ref):
    xv = x_ref[...]
    gi_ref[...] = jnp.dot(xv, wa_ref[...], preferred_element_type=f32)
    gk_ref[...] = jnp.dot(xv, wb_ref[...], preferred_element_type=f32)


def _k2_body(ea_ref, eb_ref, at_ref, w1c_ref, b1_ref, w2_ref, b2_ref,
             ha_ref, hb_ref):
    e = ea_ref[...] + eb_ref[...]
    h1 = (e + jnp.dot(at_ref[...], w1c_ref[...], preferred_element_type=f32)
          + b1_ref[...])
    h1 = h1 * jax.nn.sigmoid(h1)
    h2 = jnp.dot(h1, w2_ref[...], preferred_element_type=f32) + b2_ref[...]
    h2 = h2 * jax.nn.sigmoid(h2)
    ha_ref[...] = h2[:, 0:HIDHALF]
    hb_ref[...] = h2[:, HIDHALF:HIDDEN]


def _k3_body(s0a_ref, s1a_ref, s0b_ref, s1b_ref, c0_ref, c1_ref,
             w3a_ref, w3b_ref, b3_ref, wo_ref, o_ref):
    cnt = c0_ref[...][:, 0:1] + c1_ref[...][:, 0:1]
    recip = 1.0 / jnp.maximum(cnt, 1.0)
    mean_a = (s0a_ref[...] + s1a_ref[...]) * recip
    mean_b = (s0b_ref[...] + s1b_ref[...]) * recip
    pre = (jnp.dot(mean_a, w3a_ref[...], preferred_element_type=f32)
           + jnp.dot(mean_b, w3b_ref[...], preferred_element_type=f32)
           + b3_ref[...])
    pre = jnp.where(cnt > 0.5, pre, 0.0)
    o_ref[...] = jnp.dot(pre, wo_ref[...],
                         preferred_element_type=f32) * (1.0 / np.sqrt(128.0))


_MESH = plsc.VectorSubcoreMesh(core_axis_name="c", subcore_axis_name="s")


@functools.partial(
    pl.kernel,
    out_type=(jax.ShapeDtypeStruct((PAD_A, HIDDEN), f32),
              jax.ShapeDtypeStruct((PAD_A, HIDDEN), f32)),
    mesh=_MESH,
    scratch_types=[
        pltpu.VMEM((RPW, IDXW), jnp.int32),
        pltpu.VMEM((RPW, IDXW), jnp.int32),
        pltpu.VMEM((CH, HIDDEN), f32),
        pltpu.VMEM((CH, HIDDEN), f32),
        pltpu.SemaphoreType.DMA,
        pltpu.SemaphoreType.DMA,
    ],
    compiler_params=pltpu.CompilerParams(use_tc_tiling_on_sc=False),
)
def _s1_gather(gi_hbm, gk_hbm, i2_hbm, k2_hbm, ea_hbm, eb_hbm,
               ii_v, kk_v, ba, bb, sa, sb):
    wid = lax.axis_index("s") * NC + lax.axis_index("c")
    r0 = wid * RPW
    pltpu.sync_copy(i2_hbm.at[pl.ds(r0, RPW)], ii_v)
    pltpu.sync_copy(k2_hbm.at[pl.ds(r0, RPW)], kk_v)

    def step(g, carry):
        handles = []
        for s in range(4):
            r = g * 4 + s
            handles.append(pltpu.async_copy(
                gi_hbm.at[ii_v.at[r]], ba.at[pl.ds(s * IDXW, IDXW)], sa))
            handles.append(pltpu.async_copy(
                gk_hbm.at[kk_v.at[r]], bb.at[pl.ds(s * IDXW, IDXW)], sb))
        for h in handles:
            h.wait()
        base = (r0 + g * 4) * IDXW
        pltpu.sync_copy(ba, ea_hbm.at[pl.ds(base, CH)])
        pltpu.sync_copy(bb, eb_hbm.at[pl.ds(base, CH)])
        return carry

    lax.fori_loop(0, STEPS, step, 0)


@functools.partial(
    pl.kernel,
    out_type=(pltpu.HBM((NT, HIDHALF), f32),
              pltpu.HBM((NT, HIDHALF), f32),
              pltpu.HBM((NT, HIDHALF), f32),
              pltpu.HBM((NT, HIDHALF), f32),
              pltpu.HBM((NT, 16), f32),
              pltpu.HBM((NT, 16), f32)),
    mesh=_MESH,
    scratch_types=[
        pltpu.VMEM((RPW, IDXW), jnp.int32),
        pltpu.VMEM((CH, HIDHALF), f32),
        pltpu.VMEM((IDXW, 16), f32),
        pltpu.VMEM((STRIPE, HIDHALF), f32),
        pltpu.VMEM((STRIPE, 16), f32),
        pltpu.VMEM_SHARED((NT, HIDHALF), f32),
        pltpu.VMEM_SHARED((NT, 16), f32),
    ],
    compiler_params=pltpu.CompilerParams(use_tc_tiling_on_sc=False),
)
def _s2_scatter(j2_hbm, ha_hbm, hb_hbm, s0a_hbm, s1a_hbm, s0b_hbm, s1b_hbm,
                c0_hbm, c1_hbm, jv, hb, ones_v, zs, zc, ssum, scnt):
    cid = lax.axis_index("c")
    sid = lax.axis_index("s")
    wid = sid * NC + cid
    r0 = wid * RPW
    nr0 = sid * STRIPE
    pltpu.sync_copy(j2_hbm.at[pl.ds(r0, RPW)], jv)

    def zfill(t, carry):
        for q in range(HIDHALF // 16):
            zs[t, pl.ds(q * 16, 16)] = jnp.zeros((16,), f32)
        zc[t] = jnp.zeros((16,), f32)
        return carry

    def ofill(t, carry):
        ones_v[t] = jnp.ones((16,), f32)
        return carry

    lax.fori_loop(0, STRIPE, zfill, 0)
    lax.fori_loop(0, IDXW, ofill, 0)
    pltpu.sync_copy(zs, ssum.at[pl.ds(nr0, STRIPE)])
    pltpu.sync_copy(zc, scnt.at[pl.ds(nr0, STRIPE)])
    plsc.subcore_barrier()

    def step_a(g, carry):
        base = (r0 + g * 4) * IDXW
        pltpu.sync_copy(ha_hbm.at[pl.ds(base, CH)], hb)
        for s in range(4):
            r = g * 4 + s
            pltpu.sync_copy(hb.at[pl.ds(s * IDXW, IDXW)],
                            ssum.at[jv.at[r]], add=True)
            pltpu.sync_copy(ones_v, scnt.at[jv.at[r]], add=True)
        return carry

    lax.fori_loop(0, STEPS, step_a, 0)
    plsc.subcore_barrier()

    pltpu.sync_copy(scnt.at[pl.ds(nr0, STRIPE)], zc)

    @pl.when(cid == 0)
    def _():
        pltpu.sync_copy(zc, c0_hbm.at[pl.ds(nr0, STRIPE)])

    @pl.when(cid == 1)
    def _():
        pltpu.sync_copy(zc, c1_hbm.at[pl.ds(nr0, STRIPE)])

    pltpu.sync_copy(ssum.at[pl.ds(nr0, STRIPE)], zs)

    @pl.when(cid == 0)
    def _():
        pltpu.sync_copy(zs, s0a_hbm.at[pl.ds(nr0, STRIPE)])

    @pl.when(cid == 1)
    def _():
        pltpu.sync_copy(zs, s1a_hbm.at[pl.ds(nr0, STRIPE)])

    def zfill2(t, carry):
        for q in range(HIDHALF // 16):
            zs[t, pl.ds(q * 16, 16)] = jnp.zeros((16,), f32)
        return carry

    lax.fori_loop(0, STRIPE, zfill2, 0)
    pltpu.sync_copy(zs, ssum.at[pl.ds(nr0, STRIPE)])
    plsc.subcore_barrier()

    def step_b(g, carry):
        base = (r0 + g * 4) * IDXW
        pltpu.sync_copy(hb_hbm.at[pl.ds(base, CH)], hb)
        for s in range(4):
            r = g * 4 + s
            pltpu.sync_copy(hb.at[pl.ds(s * IDXW, IDXW)],
                            ssum.at[jv.at[r]], add=True)
        return carry

    lax.fori_loop(0, STEPS, step_b, 0)
    plsc.subcore_barrier()

    pltpu.sync_copy(ssum.at[pl.ds(nr0, STRIPE)], zs)

    @pl.when(cid == 0)
    def _():
        pltpu.sync_copy(zs, s0b_hbm.at[pl.ds(nr0, STRIPE)])

    @pl.when(cid == 1)
    def _():
        pltpu.sync_copy(zs, s1b_hbm.at[pl.ds(nr0, STRIPE)])


def kernel(x, triple_index, triple_attr, W1, b1, W2, b2, W3, b3, Wout):
    i_idx = triple_index[0]
    j_idx = triple_index[1]
    k_idx = triple_index[2]
    pad = PAD_A - N_ANGLES
    i2 = jnp.concatenate([i_idx, jnp.zeros((pad,), jnp.int32)]).reshape(ROWS, IDXW)
    k2 = jnp.concatenate([k_idx, jnp.zeros((pad,), jnp.int32)]).reshape(ROWS, IDXW)
    j2 = jnp.concatenate(
        [j_idx, jnp.full((pad,), N_NODES, jnp.int32)]).reshape(ROWS, IDXW)
    attr_p = jnp.concatenate([triple_attr, jnp.zeros((pad, 2), f32)])

    W1a = W1[0:SCALAR_DIM]
    W1b = W1[SCALAR_DIM:2 * SCALAR_DIM]
    W1c = W1[2 * SCALAR_DIM:]
    b1r = b1.reshape(1, HIDDEN)
    b2r = b2.reshape(1, HIDDEN)
    b3r = b3.reshape(1, SCALAR_DIM)

    gi, gk = pl.pallas_call(
        _k1_body,
        grid=(10,),
        in_specs=[
            pl.BlockSpec((1000, SCALAR_DIM), lambda m: (m, 0)),
            pl.BlockSpec((SCALAR_DIM, HIDDEN), lambda m: (0, 0)),
            pl.BlockSpec((SCALAR_DIM, HIDDEN), lambda m: (0, 0)),
        ],
        out_specs=[
            pl.BlockSpec((1000, HIDDEN), lambda m: (m, 0)),
            pl.BlockSpec((1000, HIDDEN), lambda m: (m, 0)),
        ],
        out_shape=[
            jax.ShapeDtypeStruct((N_NODES, HIDDEN), f32),
            jax.ShapeDtypeStruct((N_NODES, HIDDEN), f32),
        ],
    )(x, W1a, W1b)

    ea, eb = _s1_gather(gi, gk, i2, k2)

    BK2 = 1024
    h2a, h2b = pl.pallas_call(
        _k2_body,
        grid=(PAD_A // BK2,),
        in_specs=[
            pl.BlockSpec((BK2, HIDDEN), lambda m: (m, 0)),
            pl.BlockSpec((BK2, HIDDEN), lambda m: (m, 0)),
            pl.BlockSpec((BK2, 2), lambda m: (m, 0)),
            pl.BlockSpec((2, HIDDEN), lambda m: (0, 0)),
            pl.BlockSpec((1, HIDDEN), lambda m: (0, 0)),
            pl.BlockSpec((HIDDEN, HIDDEN), lambda m: (0, 0)),
            pl.BlockSpec((1, HIDDEN), lambda m: (0, 0)),
        ],
        out_specs=[
            pl.BlockSpec((BK2, HIDHALF), lambda m: (m, 0)),
            pl.BlockSpec((BK2, HIDHALF), lambda m: (m, 0)),
        ],
        out_shape=[
            jax.ShapeDtypeStruct((PAD_A, HIDHALF), f32),
            jax.ShapeDtypeStruct((PAD_A, HIDHALF), f32),
        ],
    )(ea, eb, attr_p, W1c, b1r, W2, b2r)

    s0a, s1a, s0b, s1b, c0, c1 = _s2_scatter(j2, h2a, h2b)

    BK3 = 1024
    out_p = pl.pallas_call(
        _k3_body,
        grid=(NT // BK3,),
        in_specs=[
            pl.BlockSpec((BK3, HIDHALF), lambda m: (m, 0)),
            pl.BlockSpec((BK3, HIDHALF), lambda m: (m, 0)),
            pl.BlockSpec((BK3, HIDHALF), lambda m: (m, 0)),
            pl.BlockSpec((BK3, HIDHALF), lambda m: (m, 0)),
            pl.BlockSpec((BK3, 16), lambda m: (m, 0)),
            pl.BlockSpec((BK3, 16), lambda m: (m, 0)),
            pl.BlockSpec((HIDHALF, SCALAR_DIM), lambda m: (0, 0)),
            pl.BlockSpec((HIDHALF, SCALAR_DIM), lambda m: (0, 0)),
            pl.BlockSpec((1, SCALAR_DIM), lambda m: (0, 0)),
            pl.BlockSpec((SCALAR_DIM, SCALAR_DIM), lambda m: (0, 0)),
        ],
        out_specs=pl.BlockSpec((BK3, SCALAR_DIM), lambda m: (m, 0)),
        out_shape=jax.ShapeDtypeStruct((NT, SCALAR_DIM), f32),
    )(s0a, s1a, s0b, s1b, c0, c1, W3[0:HIDHALF], W3[HIDHALF:HIDDEN], b3r, Wout)

    return out_p[:N_NODES]


# speedup vs baseline: 2.1821x; 1.0271x over previous
import functools

import jax
import jax.numpy as jnp
import numpy as np
from jax import lax
from jax.experimental import pallas as pl
from jax.experimental.pallas import tpu as pltpu
from jax.experimental.pallas import tpu_sc as plsc

N_NODES = 10000
N_ANGLES = 320000
SCALAR_DIM = 128
HIDDEN = 64

NC = 2
NS = 16
NW = NC * NS
IDXW = 128
RPW = 80
PAD_A = NW * RPW * IDXW
ROWS = PAD_A // IDXW
NT = 10240
STRIPE = NT // NS
CH = 512
STEPS = RPW // 4
HIDHALF = HIDDEN // 2

f32 = jnp.float32


def _k1_body(x_ref, wa_ref, wb_ref, gi_ref, gk_ref):
    xv = x_ref[...]
    gi_ref[...] = jnp.dot(xv, wa_ref[...], preferred_element_type=f32)
    gk_ref[...] = jnp.dot(xv, wb_ref[...], preferred_element_type=f32)


def _k2_body(ea_ref, eb_ref, at_ref, w1c_ref, b1_ref, w2_ref, b2_ref,
             ha_ref, hb_ref):
    e = ea_ref[...] + eb_ref[...]
    h1 = (e + jnp.dot(at_ref[...], w1c_ref[...], preferred_element_type=f32)
          + b1_ref[...])
    h1 = h1 * jax.nn.sigmoid(h1)
    h2 = jnp.dot(h1, w2_ref[...], preferred_element_type=f32) + b2_ref[...]
    h2 = h2 * jax.nn.sigmoid(h2)
    ha_ref[...] = h2[:, 0:HIDHALF]
    hb_ref[...] = h2[:, HIDHALF:HIDDEN]


def _k3_body(s0a_ref, s1a_ref, s0b_ref, s1b_ref, c0_ref, c1_ref,
             w3a_ref, w3b_ref, b3_ref, wo_ref, o_ref):
    cnt = c0_ref[...][:, 0:1] + c1_ref[...][:, 0:1]
    recip = 1.0 / jnp.maximum(cnt, 1.0)
    mean_a = (s0a_ref[...] + s1a_ref[...]) * recip
    mean_b = (s0b_ref[...] + s1b_ref[...]) * recip
    pre = (jnp.dot(mean_a, w3a_ref[...], preferred_element_type=f32)
           + jnp.dot(mean_b, w3b_ref[...], preferred_element_type=f32)
           + b3_ref[...])
    pre = jnp.where(cnt > 0.5, pre, 0.0)
    o_ref[...] = jnp.dot(pre, wo_ref[...],
                         preferred_element_type=f32) * (1.0 / np.sqrt(128.0))


_MESH = plsc.VectorSubcoreMesh(core_axis_name="c", subcore_axis_name="s")


S1B = 256
S1_STEPS = RPW // 2


@functools.partial(
    pl.kernel,
    out_type=(jax.ShapeDtypeStruct((PAD_A, HIDDEN), f32),
              jax.ShapeDtypeStruct((PAD_A, HIDDEN), f32)),
    mesh=_MESH,
    scratch_types=[
        pltpu.VMEM((RPW, IDXW), jnp.int32),
        pltpu.VMEM((RPW, IDXW), jnp.int32),
        pltpu.VMEM((2 * S1B, HIDDEN), f32),
        pltpu.VMEM((2 * S1B, HIDDEN), f32),
        pltpu.SemaphoreType.DMA,
        pltpu.SemaphoreType.DMA,
        pltpu.SemaphoreType.DMA,
    ],
    compiler_params=pltpu.CompilerParams(use_tc_tiling_on_sc=False),
)
def _s1_gather(gi_hbm, gk_hbm, i2_hbm, k2_hbm, ea_hbm, eb_hbm,
               ii_v, kk_v, ba, bb, sa, sb, so):
    wid = lax.axis_index("s") * NC + lax.axis_index("c")
    r0 = wid * RPW
    pltpu.sync_copy(i2_hbm.at[pl.ds(r0, RPW)], ii_v)
    pltpu.sync_copy(k2_hbm.at[pl.ds(r0, RPW)], kk_v)

    def step(g, carry):
        half = (g % 2) * S1B

        @pl.when(g >= 2)
        def _():
            pltpu.make_async_copy(ba.at[pl.ds(half, S1B)],
                                  ea_hbm.at[pl.ds(0, S1B)], so).wait()
            pltpu.make_async_copy(bb.at[pl.ds(half, S1B)],
                                  eb_hbm.at[pl.ds(0, S1B)], so).wait()

        handles = []
        for s in range(2):
            r = g * 2 + s
            handles.append(pltpu.async_copy(
                gi_hbm.at[ii_v.at[r]],
                ba.at[pl.ds(half + s * IDXW, IDXW)], sa))
            handles.append(pltpu.async_copy(
                gk_hbm.at[kk_v.at[r]],
                bb.at[pl.ds(half + s * IDXW, IDXW)], sb))
        for h in handles:
            h.wait()
        base = (r0 + g * 2) * IDXW
        pltpu.async_copy(ba.at[pl.ds(half, S1B)],
                         ea_hbm.at[pl.ds(base, S1B)], so)
        pltpu.async_copy(bb.at[pl.ds(half, S1B)],
                         eb_hbm.at[pl.ds(base, S1B)], so)
        return carry

    lax.fori_loop(0, S1_STEPS, step, 0)
    for off in (0, S1B):
        pltpu.make_async_copy(ba.at[pl.ds(off, S1B)],
                              ea_hbm.at[pl.ds(0, S1B)], so).wait()
        pltpu.make_async_copy(bb.at[pl.ds(off, S1B)],
                              eb_hbm.at[pl.ds(0, S1B)], so).wait()


@functools.partial(
    pl.kernel,
    out_type=(pltpu.HBM((NT, HIDHALF), f32),
              pltpu.HBM((NT, HIDHALF), f32),
              pltpu.HBM((NT, HIDHALF), f32),
              pltpu.HBM((NT, HIDHALF), f32),
              pltpu.HBM((NT, 16), f32),
              pltpu.HBM((NT, 16), f32)),
    mesh=_MESH,
    scratch_types=[
        pltpu.VMEM((RPW, IDXW), jnp.int32),
        pltpu.VMEM((2 * CH, HIDHALF), f32),
        pltpu.VMEM((IDXW, 16), f32),
        pltpu.VMEM((STRIPE, HIDHALF), f32),
        pltpu.VMEM((STRIPE, 16), f32),
        pltpu.VMEM_SHARED((NT, HIDHALF), f32),
        pltpu.VMEM_SHARED((NT, 16), f32),
        pltpu.SemaphoreType.DMA,
    ],
    compiler_params=pltpu.CompilerParams(use_tc_tiling_on_sc=False),
)
def _s2_scatter(j2_hbm, ha_hbm, hb_hbm, s0a_hbm, s1a_hbm, s0b_hbm, s1b_hbm,
                c0_hbm, c1_hbm, jv, hb, ones_v, zs, zc, ssum, scnt, sl):
    cid = lax.axis_index("c")
    sid = lax.axis_index("s")
    wid = sid * NC + cid
    r0 = wid * RPW
    nr0 = sid * STRIPE
    pltpu.sync_copy(j2_hbm.at[pl.ds(r0, RPW)], jv)

    def zfill(t, carry):
        for q in range(HIDHALF // 16):
            zs[t, pl.ds(q * 16, 16)] = jnp.zeros((16,), f32)
        zc[t] = jnp.zeros((16,), f32)
        return carry

    def ofill(t, carry):
        ones_v[t] = jnp.ones((16,), f32)
        return carry

    lax.fori_loop(0, STRIPE, zfill, 0)
    lax.fori_loop(0, IDXW, ofill, 0)
    pltpu.sync_copy(zs, ssum.at[pl.ds(nr0, STRIPE)])
    pltpu.sync_copy(zc, scnt.at[pl.ds(nr0, STRIPE)])
    plsc.subcore_barrier()

    pltpu.async_copy(ha_hbm.at[pl.ds(r0 * IDXW, CH)], hb.at[pl.ds(0, CH)], sl)

    def step_a(g, carry):
        half = (g % 2) * CH
        pltpu.make_async_copy(ha_hbm.at[pl.ds(0, CH)],
                              hb.at[pl.ds(half, CH)], sl).wait()

        @pl.when(g + 1 < STEPS)
        def _():
            nxt = ((g + 1) % 2) * CH
            pltpu.async_copy(ha_hbm.at[pl.ds((r0 + (g + 1) * 4) * IDXW, CH)],
                             hb.at[pl.ds(nxt, CH)], sl)

        for s in range(4):
            r = g * 4 + s
            pltpu.sync_copy(hb.at[pl.ds(half + s * IDXW, IDXW)],
                            ssum.at[jv.at[r]], add=True)
            pltpu.sync_copy(ones_v, scnt.at[jv.at[r]], add=True)
        return carry

    lax.fori_loop(0, STEPS, step_a, 0)
    plsc.subcore_barrier()

    pltpu.sync_copy(scnt.at[pl.ds(nr0, STRIPE)], zc)

    @pl.when(cid == 0)
    def _():
        pltpu.sync_copy(zc, c0_hbm.at[pl.ds(nr0, STRIPE)])

    @pl.when(cid == 1)
    def _():
        pltpu.sync_copy(zc, c1_hbm.at[pl.ds(nr0, STRIPE)])

    pltpu.sync_copy(ssum.at[pl.ds(nr0, STRIPE)], zs)

    @pl.when(cid == 0)
    def _():
        pltpu.sync_copy(zs, s0a_hbm.at[pl.ds(nr0, STRIPE)])

    @pl.when(cid == 1)
    def _():
        pltpu.sync_copy(zs, s1a_hbm.at[pl.ds(nr0, STRIPE)])

    def zfill2(t, carry):
        for q in range(HIDHALF // 16):
            zs[t, pl.ds(q * 16, 16)] = jnp.zeros((16,), f32)
        return carry

    lax.fori_loop(0, STRIPE, zfill2, 0)
    pltpu.sync_copy(zs, ssum.at[pl.ds(nr0, STRIPE)])
    plsc.subcore_barrier()

    pltpu.async_copy(hb_hbm.at[pl.ds(r0 * IDXW, CH)], hb.at[pl.ds(0, CH)], sl)

    def step_b(g, carry):
        half = (g % 2) * CH
        pltpu.make_async_copy(hb_hbm.at[pl.ds(0, CH)],
                              hb.at[pl.ds(half, CH)], sl).wait()

        @pl.when(g + 1 < STEPS)
        def _():
            nxt = ((g + 1) % 2) * CH
            pltpu.async_copy(hb_hbm.at[pl.ds((r0 + (g + 1) * 4) * IDXW, CH)],
                             hb.at[pl.ds(nxt, CH)], sl)

        for s in range(4):
            r = g * 4 + s
            pltpu.sync_copy(hb.at[pl.ds(half + s * IDXW, IDXW)],
                            ssum.at[jv.at[r]], add=True)
        return carry

    lax.fori_loop(0, STEPS, step_b, 0)
    plsc.subcore_barrier()

    pltpu.sync_copy(ssum.at[pl.ds(nr0, STRIPE)], zs)

    @pl.when(cid == 0)
    def _():
        pltpu.sync_copy(zs, s0b_hbm.at[pl.ds(nr0, STRIPE)])

    @pl.when(cid == 1)
    def _():
        pltpu.sync_copy(zs, s1b_hbm.at[pl.ds(nr0, STRIPE)])


def kernel(x, triple_index, triple_attr, W1, b1, W2, b2, W3, b3, Wout):
    i_idx = triple_index[0]
    j_idx = triple_index[1]
    k_idx = triple_index[2]
    pad = PAD_A - N_ANGLES
    i2 = jnp.concatenate([i_idx, jnp.zeros((pad,), jnp.int32)]).reshape(ROWS, IDXW)
    k2 = jnp.concatenate([k_idx, jnp.zeros((pad,), jnp.int32)]).reshape(ROWS, IDXW)
    j2 = jnp.concatenate(
        [j_idx, jnp.full((pad,), N_NODES, jnp.int32)]).reshape(ROWS, IDXW)
    attr_p = jnp.concatenate([triple_attr, jnp.zeros((pad, 2), f32)])

    W1a = W1[0:SCALAR_DIM]
    W1b = W1[SCALAR_DIM:2 * SCALAR_DIM]
    W1c = W1[2 * SCALAR_DIM:]
    b1r = b1.reshape(1, HIDDEN)
    b2r = b2.reshape(1, HIDDEN)
    b3r = b3.reshape(1, SCALAR_DIM)

    gi, gk = pl.pallas_call(
        _k1_body,
        grid=(10,),
        in_specs=[
            pl.BlockSpec((1000, SCALAR_DIM), lambda m: (m, 0)),
            pl.BlockSpec((SCALAR_DIM, HIDDEN), lambda m: (0, 0)),
            pl.BlockSpec((SCALAR_DIM, HIDDEN), lambda m: (0, 0)),
        ],
        out_specs=[
            pl.BlockSpec((1000, HIDDEN), lambda m: (m, 0)),
            pl.BlockSpec((1000, HIDDEN), lambda m: (m, 0)),
        ],
        out_shape=[
            jax.ShapeDtypeStruct((N_NODES, HIDDEN), f32),
            jax.ShapeDtypeStruct((N_NODES, HIDDEN), f32),
        ],
    )(x, W1a, W1b)

    ea, eb = _s1_gather(gi, gk, i2, k2)

    BK2 = 1024
    h2a, h2b = pl.pallas_call(
        _k2_body,
        grid=(PAD_A // BK2,),
        in_specs=[
            pl.BlockSpec((BK2, HIDDEN), lambda m: (m, 0)),
            pl.BlockSpec((BK2, HIDDEN), lambda m: (m, 0)),
            pl.BlockSpec((BK2, 2), lambda m: (m, 0)),
            pl.BlockSpec((2, HIDDEN), lambda m: (0, 0)),
            pl.BlockSpec((1, HIDDEN), lambda m: (0, 0)),
            pl.BlockSpec((HIDDEN, HIDDEN), lambda m: (0, 0)),
            pl.BlockSpec((1, HIDDEN), lambda m: (0, 0)),
        ],
        out_specs=[
            pl.BlockSpec((BK2, HIDHALF), lambda m: (m, 0)),
            pl.BlockSpec((BK2, HIDHALF), lambda m: (m, 0)),
        ],
        out_shape=[
            jax.ShapeDtypeStruct((PAD_A, HIDHALF), f32),
            jax.ShapeDtypeStruct((PAD_A, HIDHALF), f32),
        ],
    )(ea, eb, attr_p, W1c, b1r, W2, b2r)

    s0a, s1a, s0b, s1b, c0, c1 = _s2_scatter(j2, h2a, h2b)

    BK3 = 1024
    out_p = pl.pallas_call(
        _k3_body,
        grid=(NT // BK3,),
        in_specs=[
            pl.BlockSpec((BK3, HIDHALF), lambda m: (m, 0)),
            pl.BlockSpec((BK3, HIDHALF), lambda m: (m, 0)),
            pl.BlockSpec((BK3, HIDHALF), lambda m: (m, 0)),
            pl.BlockSpec((BK3, HIDHALF), lambda m: (m, 0)),
            pl.BlockSpec((BK3, 16), lambda m: (m, 0)),
            pl.BlockSpec((BK3, 16), lambda m: (m, 0)),
            pl.BlockSpec((HIDHALF, SCALAR_DIM), lambda m: (0, 0)),
            pl.BlockSpec((HIDHALF, SCALAR_DIM), lambda m: (0, 0)),
            pl.BlockSpec((1, SCALAR_DIM), lambda m: (0, 0)),
            pl.BlockSpec((SCALAR_DIM, SCALAR_DIM), lambda m: (0, 0)),
        ],
        out_specs=pl.BlockSpec((BK3, SCALAR_DIM), lambda m: (m, 0)),
        out_shape=jax.ShapeDtypeStruct((NT, SCALAR_DIM), f32),
    )(s0a, s1a, s0b, s1b, c0, c1, W3[0:HIDHALF], W3[HIDHALF:HIDDEN], b3r, Wout)

    return out_p[:N_NODES]


# speedup vs baseline: 2.1897x; 1.0035x over previous
import functools

import jax
import jax.numpy as jnp
import numpy as np
from jax import lax
from jax.experimental import pallas as pl
from jax.experimental.pallas import tpu as pltpu
from jax.experimental.pallas import tpu_sc as plsc

N_NODES = 10000
N_ANGLES = 320000
SCALAR_DIM = 128
HIDDEN = 64

NC = 2
NS = 16
NW = NC * NS
IDXW = 128
RPW = 80
PAD_A = NW * RPW * IDXW
ROWS = PAD_A // IDXW
NT = 10240
STRIPE = NT // NS
CH = 512
STEPS = RPW // 4
HIDHALF = HIDDEN // 2

f32 = jnp.float32
bf16 = jnp.bfloat16


def _k1_body(x_ref, wa_ref, wb_ref, gi_ref, gk_ref):
    xv = x_ref[...]
    gi_ref[...] = jnp.dot(
        xv, wa_ref[...], preferred_element_type=f32).astype(bf16)
    gk_ref[...] = jnp.dot(
        xv, wb_ref[...], preferred_element_type=f32).astype(bf16)


def _k2_body(ea_ref, eb_ref, at_ref, w1c_ref, b1_ref, w2_ref, b2_ref,
             ha_ref, hb_ref):
    e = ea_ref[...].astype(f32) + eb_ref[...].astype(f32)
    h1 = (e + jnp.dot(at_ref[...], w1c_ref[...], preferred_element_type=f32)
          + b1_ref[...])
    h1 = h1 * jax.nn.sigmoid(h1)
    h2 = jnp.dot(h1, w2_ref[...], preferred_element_type=f32) + b2_ref[...]
    h2 = h2 * jax.nn.sigmoid(h2)
    ha_ref[...] = h2[:, 0:HIDHALF]
    hb_ref[...] = h2[:, HIDHALF:HIDDEN]


def _k3_body(s0a_ref, s1a_ref, s0b_ref, s1b_ref, c0_ref, c1_ref,
             w3a_ref, w3b_ref, b3_ref, wo_ref, o_ref):
    cnt = c0_ref[...][:, 0:1] + c1_ref[...][:, 0:1]
    recip = 1.0 / jnp.maximum(cnt, 1.0)
    mean_a = (s0a_ref[...] + s1a_ref[...]) * recip
    mean_b = (s0b_ref[...] + s1b_ref[...]) * recip
    pre = (jnp.dot(mean_a, w3a_ref[...], preferred_element_type=f32)
           + jnp.dot(mean_b, w3b_ref[...], preferred_element_type=f32)
           + b3_ref[...])
    pre = jnp.where(cnt > 0.5, pre, 0.0)
    o_ref[...] = jnp.dot(pre, wo_ref[...],
                         preferred_element_type=f32) * (1.0 / np.sqrt(128.0))


_MESH = plsc.VectorSubcoreMesh(core_axis_name="c", subcore_axis_name="s")


S1B = 256
S1_STEPS = RPW // 2


@functools.partial(
    pl.kernel,
    out_type=(jax.ShapeDtypeStruct((PAD_A, HIDDEN), bf16),
              jax.ShapeDtypeStruct((PAD_A, HIDDEN), bf16)),
    mesh=_MESH,
    scratch_types=[
        pltpu.VMEM((RPW, IDXW), jnp.int32),
        pltpu.VMEM((RPW, IDXW), jnp.int32),
        pltpu.VMEM((2 * S1B, HIDDEN), bf16),
        pltpu.VMEM((2 * S1B, HIDDEN), bf16),
        pltpu.SemaphoreType.DMA,
        pltpu.SemaphoreType.DMA,
        pltpu.SemaphoreType.DMA,
    ],
    compiler_params=pltpu.CompilerParams(use_tc_tiling_on_sc=False),
)
def _s1_gather(gi_hbm, gk_hbm, i2_hbm, k2_hbm, ea_hbm, eb_hbm,
               ii_v, kk_v, ba, bb, sa, sb, so):
    wid = lax.axis_index("s") * NC + lax.axis_index("c")
    r0 = wid * RPW
    pltpu.sync_copy(i2_hbm.at[pl.ds(r0, RPW)], ii_v)
    pltpu.sync_copy(k2_hbm.at[pl.ds(r0, RPW)], kk_v)

    def step(g, carry):
        half = (g % 2) * S1B

        @pl.when(g >= 2)
        def _():
            pltpu.make_async_copy(ba.at[pl.ds(half, S1B)],
                                  ea_hbm.at[pl.ds(0, S1B)], so).wait()
            pltpu.make_async_copy(bb.at[pl.ds(half, S1B)],
                                  eb_hbm.at[pl.ds(0, S1B)], so).wait()

        handles = []
        for s in range(2):
            r = g * 2 + s
            handles.append(pltpu.async_copy(
                gi_hbm.at[ii_v.at[r]],
                ba.at[pl.ds(half + s * IDXW, IDXW)], sa))
            handles.append(pltpu.async_copy(
                gk_hbm.at[kk_v.at[r]],
                bb.at[pl.ds(half + s * IDXW, IDXW)], sb))
        for h in handles:
            h.wait()
        base = (r0 + g * 2) * IDXW
        pltpu.async_copy(ba.at[pl.ds(half, S1B)],
                         ea_hbm.at[pl.ds(base, S1B)], so)
        pltpu.async_copy(bb.at[pl.ds(half, S1B)],
                         eb_hbm.at[pl.ds(base, S1B)], so)
        return carry

    lax.fori_loop(0, S1_STEPS, step, 0)
    for off in (0, S1B):
        pltpu.make_async_copy(ba.at[pl.ds(off, S1B)],
                              ea_hbm.at[pl.ds(0, S1B)], so).wait()
        pltpu.make_async_copy(bb.at[pl.ds(off, S1B)],
                              eb_hbm.at[pl.ds(0, S1B)], so).wait()


@functools.partial(
    pl.kernel,
    out_type=(pltpu.HBM((NT, HIDHALF), f32),
              pltpu.HBM((NT, HIDHALF), f32),
              pltpu.HBM((NT, HIDHALF), f32),
              pltpu.HBM((NT, HIDHALF), f32),
              pltpu.HBM((NT, 16), f32),
              pltpu.HBM((NT, 16), f32)),
    mesh=_MESH,
    scratch_types=[
        pltpu.VMEM((RPW, IDXW), jnp.int32),
        pltpu.VMEM((2 * CH, HIDHALF), f32),
        pltpu.VMEM((IDXW, 16), f32),
        pltpu.VMEM((STRIPE, HIDHALF), f32),
        pltpu.VMEM((STRIPE, 16), f32),
        pltpu.VMEM_SHARED((NT, HIDHALF), f32),
        pltpu.VMEM_SHARED((NT, 16), f32),
        pltpu.SemaphoreType.DMA,
    ],
    compiler_params=pltpu.CompilerParams(use_tc_tiling_on_sc=False),
)
def _s2_scatter(j2_hbm, ha_hbm, hb_hbm, s0a_hbm, s1a_hbm, s0b_hbm, s1b_hbm,
                c0_hbm, c1_hbm, jv, hb, ones_v, zs, zc, ssum, scnt, sl):
    cid = lax.axis_index("c")
    sid = lax.axis_index("s")
    wid = sid * NC + cid
    r0 = wid * RPW
    nr0 = sid * STRIPE
    pltpu.sync_copy(j2_hbm.at[pl.ds(r0, RPW)], jv)

    def zfill(t, carry):
        for q in range(HIDHALF // 16):
            zs[t, pl.ds(q * 16, 16)] = jnp.zeros((16,), f32)
        zc[t] = jnp.zeros((16,), f32)
        return carry

    def ofill(t, carry):
        ones_v[t] = jnp.ones((16,), f32)
        return carry

    lax.fori_loop(0, STRIPE, zfill, 0)
    lax.fori_loop(0, IDXW, ofill, 0)
    pltpu.sync_copy(zs, ssum.at[pl.ds(nr0, STRIPE)])
    pltpu.sync_copy(zc, scnt.at[pl.ds(nr0, STRIPE)])
    plsc.subcore_barrier()

    pltpu.async_copy(ha_hbm.at[pl.ds(r0 * IDXW, CH)], hb.at[pl.ds(0, CH)], sl)

    def step_a(g, carry):
        half = (g % 2) * CH
        pltpu.make_async_copy(ha_hbm.at[pl.ds(0, CH)],
                              hb.at[pl.ds(half, CH)], sl).wait()

        @pl.when(g + 1 < STEPS)
        def _():
            nxt = ((g + 1) % 2) * CH
            pltpu.async_copy(ha_hbm.at[pl.ds((r0 + (g + 1) * 4) * IDXW, CH)],
                             hb.at[pl.ds(nxt, CH)], sl)

        for s in range(4):
            r = g * 4 + s
            pltpu.sync_copy(hb.at[pl.ds(half + s * IDXW, IDXW)],
                            ssum.at[jv.at[r]], add=True)
            pltpu.sync_copy(ones_v, scnt.at[jv.at[r]], add=True)
        return carry

    lax.fori_loop(0, STEPS, step_a, 0)
    plsc.subcore_barrier()

    pltpu.sync_copy(scnt.at[pl.ds(nr0, STRIPE)], zc)

    @pl.when(cid == 0)
    def _():
        pltpu.sync_copy(zc, c0_hbm.at[pl.ds(nr0, STRIPE)])

    @pl.when(cid == 1)
    def _():
        pltpu.sync_copy(zc, c1_hbm.at[pl.ds(nr0, STRIPE)])

    pltpu.sync_copy(ssum.at[pl.ds(nr0, STRIPE)], zs)

    @pl.when(cid == 0)
    def _():
        pltpu.sync_copy(zs, s0a_hbm.at[pl.ds(nr0, STRIPE)])

    @pl.when(cid == 1)
    def _():
        pltpu.sync_copy(zs, s1a_hbm.at[pl.ds(nr0, STRIPE)])

    def zfill2(t, carry):
        for q in range(HIDHALF // 16):
            zs[t, pl.ds(q * 16, 16)] = jnp.zeros((16,), f32)
        return carry

    lax.fori_loop(0, STRIPE, zfill2, 0)
    pltpu.sync_copy(zs, ssum.at[pl.ds(nr0, STRIPE)])
    plsc.subcore_barrier()

    pltpu.async_copy(hb_hbm.at[pl.ds(r0 * IDXW, CH)], hb.at[pl.ds(0, CH)], sl)

    def step_b(g, carry):
        half = (g % 2) * CH
        pltpu.make_async_copy(hb_hbm.at[pl.ds(0, CH)],
                              hb.at[pl.ds(half, CH)], sl).wait()

        @pl.when(g + 1 < STEPS)
        def _():
            nxt = ((g + 1) % 2) * CH
            pltpu.async_copy(hb_hbm.at[pl.ds((r0 + (g + 1) * 4) * IDXW, CH)],
                             hb.at[pl.ds(nxt, CH)], sl)

        for s in range(4):
            r = g * 4 + s
            pltpu.sync_copy(hb.at[pl.ds(half + s * IDXW, IDXW)],
                            ssum.at[jv.at[r]], add=True)
        return carry

    lax.fori_loop(0, STEPS, step_b, 0)
    plsc.subcore_barrier()

    pltpu.sync_copy(ssum.at[pl.ds(nr0, STRIPE)], zs)

    @pl.when(cid == 0)
    def _():
        pltpu.sync_copy(zs, s0b_hbm.at[pl.ds(nr0, STRIPE)])

    @pl.when(cid == 1)
    def _():
        pltpu.sync_copy(zs, s1b_hbm.at[pl.ds(nr0, STRIPE)])


def kernel(x, triple_index, triple_attr, W1, b1, W2, b2, W3, b3, Wout):
    i_idx = triple_index[0]
    j_idx = triple_index[1]
    k_idx = triple_index[2]
    pad = PAD_A - N_ANGLES
    i2 = jnp.concatenate([i_idx, jnp.zeros((pad,), jnp.int32)]).reshape(ROWS, IDXW)
    k2 = jnp.concatenate([k_idx, jnp.zeros((pad,), jnp.int32)]).reshape(ROWS, IDXW)
    j2 = jnp.concatenate(
        [j_idx, jnp.full((pad,), N_NODES, jnp.int32)]).reshape(ROWS, IDXW)
    attr_p = jnp.concatenate([triple_attr, jnp.zeros((pad, 2), f32)])

    W1a = W1[0:SCALAR_DIM]
    W1b = W1[SCALAR_DIM:2 * SCALAR_DIM]
    W1c = W1[2 * SCALAR_DIM:]
    b1r = b1.reshape(1, HIDDEN)
    b2r = b2.reshape(1, HIDDEN)
    b3r = b3.reshape(1, SCALAR_DIM)

    gi, gk = pl.pallas_call(
        _k1_body,
        grid=(10,),
        in_specs=[
            pl.BlockSpec((1000, SCALAR_DIM), lambda m: (m, 0)),
            pl.BlockSpec((SCALAR_DIM, HIDDEN), lambda m: (0, 0)),
            pl.BlockSpec((SCALAR_DIM, HIDDEN), lambda m: (0, 0)),
        ],
        out_specs=[
            pl.BlockSpec((1000, HIDDEN), lambda m: (m, 0)),
            pl.BlockSpec((1000, HIDDEN), lambda m: (m, 0)),
        ],
        out_shape=[
            jax.ShapeDtypeStruct((N_NODES, HIDDEN), bf16),
            jax.ShapeDtypeStruct((N_NODES, HIDDEN), bf16),
        ],
    )(x, W1a, W1b)

    ea, eb = _s1_gather(gi, gk, i2, k2)

    BK2 = 1024
    h2a, h2b = pl.pallas_call(
        _k2_body,
        grid=(PAD_A // BK2,),
        in_specs=[
            pl.BlockSpec((BK2, HIDDEN), lambda m: (m, 0)),
            pl.BlockSpec((BK2, HIDDEN), lambda m: (m, 0)),
            pl.BlockSpec((BK2, 2), lambda m: (m, 0)),
            pl.BlockSpec((2, HIDDEN), lambda m: (0, 0)),
            pl.BlockSpec((1, HIDDEN), lambda m: (0, 0)),
            pl.BlockSpec((HIDDEN, HIDDEN), lambda m: (0, 0)),
            pl.BlockSpec((1, HIDDEN), lambda m: (0, 0)),
        ],
        out_specs=[
            pl.BlockSpec((BK2, HIDHALF), lambda m: (m, 0)),
            pl.BlockSpec((BK2, HIDHALF), lambda m: (m, 0)),
        ],
        out_shape=[
            jax.ShapeDtypeStruct((PAD_A, HIDHALF), f32),
            jax.ShapeDtypeStruct((PAD_A, HIDHALF), f32),
        ],
    )(ea, eb, attr_p, W1c, b1r, W2, b2r)

    s0a, s1a, s0b, s1b, c0, c1 = _s2_scatter(j2, h2a, h2b)

    BK3 = 1024
    out_p = pl.pallas_call(
        _k3_body,
        grid=(NT // BK3,),
        in_specs=[
            pl.BlockSpec((BK3, HIDHALF), lambda m: (m, 0)),
            pl.BlockSpec((BK3, HIDHALF), lambda m: (m, 0)),
            pl.BlockSpec((BK3, HIDHALF), lambda m: (m, 0)),
            pl.BlockSpec((BK3, HIDHALF), lambda m: (m, 0)),
            pl.BlockSpec((BK3, 16), lambda m: (m, 0)),
            pl.BlockSpec((BK3, 16), lambda m: (m, 0)),
            pl.BlockSpec((HIDHALF, SCALAR_DIM), lambda m: (0, 0)),
            pl.BlockSpec((HIDHALF, SCALAR_DIM), lambda m: (0, 0)),
            pl.BlockSpec((1, SCALAR_DIM), lambda m: (0, 0)),
            pl.BlockSpec((SCALAR_DIM, SCALAR_DIM), lambda m: (0, 0)),
        ],
        out_specs=pl.BlockSpec((BK3, SCALAR_DIM), lambda m: (m, 0)),
        out_shape=jax.ShapeDtypeStruct((NT, SCALAR_DIM), f32),
    )(s0a, s1a, s0b, s1b, c0, c1, W3[0:HIDHALF], W3[HIDHALF:HIDDEN], b3r, Wout)

    return out_p[:N_NODES]


# speedup vs baseline: 2.2125x; 1.0104x over previous
import functools

import jax
import jax.numpy as jnp
import numpy as np
from jax import lax
from jax.experimental import pallas as pl
from jax.experimental.pallas import tpu as pltpu
from jax.experimental.pallas import tpu_sc as plsc

N_NODES = 10000
N_ANGLES = 320000
SCALAR_DIM = 128
HIDDEN = 64

NC = 2
NS = 16
NW = NC * NS
IDXW = 128
RPW = 80
PAD_A = NW * RPW * IDXW
ROWS = PAD_A // IDXW
NT = 10240
STRIPE = NT // NS
CH = 512
STEPS = RPW // 4
HIDHALF = HIDDEN // 2

f32 = jnp.float32
bf16 = jnp.bfloat16


def _k1_body(x_ref, wa_ref, wb_ref, gi_ref, gk_ref):
    xv = x_ref[...]
    gi_ref[...] = jnp.dot(
        xv, wa_ref[...], preferred_element_type=f32).astype(bf16)
    gk_ref[...] = jnp.dot(
        xv, wb_ref[...], preferred_element_type=f32).astype(bf16)


def _k2_body(ea_ref, eb_ref, at_ref, w1c_ref, b1_ref, w2_ref, b2_ref,
             ha_ref, hb_ref):
    e = ea_ref[...].astype(f32) + eb_ref[...].astype(f32)
    h1 = (e + jnp.dot(at_ref[...], w1c_ref[...], preferred_element_type=f32)
          + b1_ref[...])
    h1 = h1 * jax.nn.sigmoid(h1)
    h2 = jnp.dot(h1, w2_ref[...], preferred_element_type=f32) + b2_ref[...]
    h2 = h2 * jax.nn.sigmoid(h2)
    ha_ref[...] = h2[:, 0:HIDHALF]
    hb_ref[...] = h2[:, HIDHALF:HIDDEN]


def _k3_body(s0a_ref, s1a_ref, s0b_ref, s1b_ref, c0_ref, c1_ref,
             w3a_ref, w3b_ref, b3_ref, wo_ref, o_ref):
    cnt = c0_ref[...][:, 0:1] + c1_ref[...][:, 0:1]
    recip = 1.0 / jnp.maximum(cnt, 1.0)
    mean_a = (s0a_ref[...] + s1a_ref[...]) * recip
    mean_b = (s0b_ref[...] + s1b_ref[...]) * recip
    pre = (jnp.dot(mean_a, w3a_ref[...], preferred_element_type=f32)
           + jnp.dot(mean_b, w3b_ref[...], preferred_element_type=f32)
           + b3_ref[...])
    pre = jnp.where(cnt > 0.5, pre, 0.0)
    o_ref[...] = jnp.dot(pre, wo_ref[...],
                         preferred_element_type=f32) * (1.0 / np.sqrt(128.0))


_MESH = plsc.VectorSubcoreMesh(core_axis_name="c", subcore_axis_name="s")


S1B = 512
S1R = S1B // IDXW
S1_STEPS = RPW // S1R


@functools.partial(
    pl.kernel,
    out_type=(jax.ShapeDtypeStruct((PAD_A, HIDDEN), bf16),
              jax.ShapeDtypeStruct((PAD_A, HIDDEN), bf16)),
    mesh=_MESH,
    scratch_types=[
        pltpu.VMEM((RPW, IDXW), jnp.int32),
        pltpu.VMEM((RPW, IDXW), jnp.int32),
        pltpu.VMEM((2 * S1B, HIDDEN), bf16),
        pltpu.VMEM((2 * S1B, HIDDEN), bf16),
        pltpu.SemaphoreType.DMA,
        pltpu.SemaphoreType.DMA,
        pltpu.SemaphoreType.DMA,
    ],
    compiler_params=pltpu.CompilerParams(use_tc_tiling_on_sc=False),
)
def _s1_gather(gi_hbm, gk_hbm, i2_hbm, k2_hbm, ea_hbm, eb_hbm,
               ii_v, kk_v, ba, bb, sa, sb, so):
    wid = lax.axis_index("s") * NC + lax.axis_index("c")
    r0 = wid * RPW
    pltpu.sync_copy(i2_hbm.at[pl.ds(r0, RPW)], ii_v)
    pltpu.sync_copy(k2_hbm.at[pl.ds(r0, RPW)], kk_v)

    def step(g, carry):
        half = (g % 2) * S1B

        @pl.when(g >= 2)
        def _():
            pltpu.make_async_copy(ba.at[pl.ds(half, S1B)],
                                  ea_hbm.at[pl.ds(0, S1B)], so).wait()
            pltpu.make_async_copy(bb.at[pl.ds(half, S1B)],
                                  eb_hbm.at[pl.ds(0, S1B)], so).wait()

        handles = []
        for s in range(S1R):
            r = g * S1R + s
            handles.append(pltpu.async_copy(
                gi_hbm.at[ii_v.at[r]],
                ba.at[pl.ds(half + s * IDXW, IDXW)], sa))
            handles.append(pltpu.async_copy(
                gk_hbm.at[kk_v.at[r]],
                bb.at[pl.ds(half + s * IDXW, IDXW)], sb))
        for h in handles:
            h.wait()
        base = (r0 + g * S1R) * IDXW
        pltpu.async_copy(ba.at[pl.ds(half, S1B)],
                         ea_hbm.at[pl.ds(base, S1B)], so)
        pltpu.async_copy(bb.at[pl.ds(half, S1B)],
                         eb_hbm.at[pl.ds(base, S1B)], so)
        return carry

    lax.fori_loop(0, S1_STEPS, step, 0)
    for off in (0, S1B):
        pltpu.make_async_copy(ba.at[pl.ds(off, S1B)],
                              ea_hbm.at[pl.ds(0, S1B)], so).wait()
        pltpu.make_async_copy(bb.at[pl.ds(off, S1B)],
                              eb_hbm.at[pl.ds(0, S1B)], so).wait()


@functools.partial(
    pl.kernel,
    out_type=(pltpu.HBM((NT, HIDHALF), f32),
              pltpu.HBM((NT, HIDHALF), f32),
              pltpu.HBM((NT, HIDHALF), f32),
              pltpu.HBM((NT, HIDHALF), f32),
              pltpu.HBM((NT, 16), f32),
              pltpu.HBM((NT, 16), f32)),
    mesh=_MESH,
    scratch_types=[
        pltpu.VMEM((RPW, IDXW), jnp.int32),
        pltpu.VMEM((2 * CH, HIDHALF), f32),
        pltpu.VMEM((IDXW, 16), f32),
        pltpu.VMEM((STRIPE, HIDHALF), f32),
        pltpu.VMEM((STRIPE, 16), f32),
        pltpu.VMEM_SHARED((NT, HIDHALF), f32),
        pltpu.VMEM_SHARED((NT, 16), f32),
        pltpu.SemaphoreType.DMA,
    ],
    compiler_params=pltpu.CompilerParams(use_tc_tiling_on_sc=False),
)
def _s2_scatter(j2_hbm, ha_hbm, hb_hbm, s0a_hbm, s1a_hbm, s0b_hbm, s1b_hbm,
                c0_hbm, c1_hbm, jv, hb, ones_v, zs, zc, ssum, scnt, sl):
    cid = lax.axis_index("c")
    sid = lax.axis_index("s")
    wid = sid * NC + cid
    r0 = wid * RPW
    nr0 = sid * STRIPE
    pltpu.sync_copy(j2_hbm.at[pl.ds(r0, RPW)], jv)

    def zfill(t, carry):
        for q in range(HIDHALF // 16):
            zs[t, pl.ds(q * 16, 16)] = jnp.zeros((16,), f32)
        zc[t] = jnp.zeros((16,), f32)
        return carry

    def ofill(t, carry):
        ones_v[t] = jnp.ones((16,), f32)
        return carry

    lax.fori_loop(0, STRIPE, zfill, 0)
    lax.fori_loop(0, IDXW, ofill, 0)
    pltpu.sync_copy(zs, ssum.at[pl.ds(nr0, STRIPE)])
    pltpu.sync_copy(zc, scnt.at[pl.ds(nr0, STRIPE)])
    plsc.subcore_barrier()

    pltpu.async_copy(ha_hbm.at[pl.ds(r0 * IDXW, CH)], hb.at[pl.ds(0, CH)], sl)

    def step_a(g, carry):
        half = (g % 2) * CH
        pltpu.make_async_copy(ha_hbm.at[pl.ds(0, CH)],
                              hb.at[pl.ds(half, CH)], sl).wait()

        @pl.when(g + 1 < STEPS)
        def _():
            nxt = ((g + 1) % 2) * CH
            pltpu.async_copy(ha_hbm.at[pl.ds((r0 + (g + 1) * 4) * IDXW, CH)],
                             hb.at[pl.ds(nxt, CH)], sl)

        for s in range(4):
            r = g * 4 + s
            pltpu.sync_copy(hb.at[pl.ds(half + s * IDXW, IDXW)],
                            ssum.at[jv.at[r]], add=True)
            pltpu.sync_copy(ones_v, scnt.at[jv.at[r]], add=True)
        return carry

    lax.fori_loop(0, STEPS, step_a, 0)
    plsc.subcore_barrier()

    pltpu.sync_copy(scnt.at[pl.ds(nr0, STRIPE)], zc)

    @pl.when(cid == 0)
    def _():
        pltpu.sync_copy(zc, c0_hbm.at[pl.ds(nr0, STRIPE)])

    @pl.when(cid == 1)
    def _():
        pltpu.sync_copy(zc, c1_hbm.at[pl.ds(nr0, STRIPE)])

    pltpu.sync_copy(ssum.at[pl.ds(nr0, STRIPE)], zs)

    @pl.when(cid == 0)
    def _():
        pltpu.sync_copy(zs, s0a_hbm.at[pl.ds(nr0, STRIPE)])

    @pl.when(cid == 1)
    def _():
        pltpu.sync_copy(zs, s1a_hbm.at[pl.ds(nr0, STRIPE)])

    def zfill2(t, carry):
        for q in range(HIDHALF // 16):
            zs[t, pl.ds(q * 16, 16)] = jnp.zeros((16,), f32)
        return carry

    lax.fori_loop(0, STRIPE, zfill2, 0)
    pltpu.sync_copy(zs, ssum.at[pl.ds(nr0, STRIPE)])
    plsc.subcore_barrier()

    pltpu.async_copy(hb_hbm.at[pl.ds(r0 * IDXW, CH)], hb.at[pl.ds(0, CH)], sl)

    def step_b(g, carry):
        half = (g % 2) * CH
        pltpu.make_async_copy(hb_hbm.at[pl.ds(0, CH)],
                              hb.at[pl.ds(half, CH)], sl).wait()

        @pl.when(g + 1 < STEPS)
        def _():
            nxt = ((g + 1) % 2) * CH
            pltpu.async_copy(hb_hbm.at[pl.ds((r0 + (g + 1) * 4) * IDXW, CH)],
                             hb.at[pl.ds(nxt, CH)], sl)

        for s in range(4):
            r = g * 4 + s
            pltpu.sync_copy(hb.at[pl.ds(half + s * IDXW, IDXW)],
                            ssum.at[jv.at[r]], add=True)
        return carry

    lax.fori_loop(0, STEPS, step_b, 0)
    plsc.subcore_barrier()

    pltpu.sync_copy(ssum.at[pl.ds(nr0, STRIPE)], zs)

    @pl.when(cid == 0)
    def _():
        pltpu.sync_copy(zs, s0b_hbm.at[pl.ds(nr0, STRIPE)])

    @pl.when(cid == 1)
    def _():
        pltpu.sync_copy(zs, s1b_hbm.at[pl.ds(nr0, STRIPE)])


def kernel(x, triple_index, triple_attr, W1, b1, W2, b2, W3, b3, Wout):
    i_idx = triple_index[0]
    j_idx = triple_index[1]
    k_idx = triple_index[2]
    pad = PAD_A - N_ANGLES
    i2 = jnp.concatenate([i_idx, jnp.zeros((pad,), jnp.int32)]).reshape(ROWS, IDXW)
    k2 = jnp.concatenate([k_idx, jnp.zeros((pad,), jnp.int32)]).reshape(ROWS, IDXW)
    j2 = jnp.concatenate(
        [j_idx, jnp.full((pad,), N_NODES, jnp.int32)]).reshape(ROWS, IDXW)
    attr_p = jnp.concatenate([triple_attr, jnp.zeros((pad, 2), f32)])

    W1a = W1[0:SCALAR_DIM]
    W1b = W1[SCALAR_DIM:2 * SCALAR_DIM]
    W1c = W1[2 * SCALAR_DIM:]
    b1r = b1.reshape(1, HIDDEN)
    b2r = b2.reshape(1, HIDDEN)
    b3r = b3.reshape(1, SCALAR_DIM)

    gi, gk = pl.pallas_call(
        _k1_body,
        grid=(10,),
        in_specs=[
            pl.BlockSpec((1000, SCALAR_DIM), lambda m: (m, 0)),
            pl.BlockSpec((SCALAR_DIM, HIDDEN), lambda m: (0, 0)),
            pl.BlockSpec((SCALAR_DIM, HIDDEN), lambda m: (0, 0)),
        ],
        out_specs=[
            pl.BlockSpec((1000, HIDDEN), lambda m: (m, 0)),
            pl.BlockSpec((1000, HIDDEN), lambda m: (m, 0)),
        ],
        out_shape=[
            jax.ShapeDtypeStruct((N_NODES, HIDDEN), bf16),
            jax.ShapeDtypeStruct((N_NODES, HIDDEN), bf16),
        ],
    )(x, W1a, W1b)

    ea, eb = _s1_gather(gi, gk, i2, k2)

    BK2 = 1024
    h2a, h2b = pl.pallas_call(
        _k2_body,
        grid=(PAD_A // BK2,),
        in_specs=[
            pl.BlockSpec((BK2, HIDDEN), lambda m: (m, 0)),
            pl.BlockSpec((BK2, HIDDEN), lambda m: (m, 0)),
            pl.BlockSpec((BK2, 2), lambda m: (m, 0)),
            pl.BlockSpec((2, HIDDEN), lambda m: (0, 0)),
            pl.BlockSpec((1, HIDDEN), lambda m: (0, 0)),
            pl.BlockSpec((HIDDEN, HIDDEN), lambda m: (0, 0)),
            pl.BlockSpec((1, HIDDEN), lambda m: (0, 0)),
        ],
        out_specs=[
            pl.BlockSpec((BK2, HIDHALF), lambda m: (m, 0)),
            pl.BlockSpec((BK2, HIDHALF), lambda m: (m, 0)),
        ],
        out_shape=[
            jax.ShapeDtypeStruct((PAD_A, HIDHALF), f32),
            jax.ShapeDtypeStruct((PAD_A, HIDHALF), f32),
        ],
    )(ea, eb, attr_p, W1c, b1r, W2, b2r)

    s0a, s1a, s0b, s1b, c0, c1 = _s2_scatter(j2, h2a, h2b)

    BK3 = 1024
    out_p = pl.pallas_call(
        _k3_body,
        grid=(NT // BK3,),
        in_specs=[
            pl.BlockSpec((BK3, HIDHALF), lambda m: (m, 0)),
            pl.BlockSpec((BK3, HIDHALF), lambda m: (m, 0)),
            pl.BlockSpec((BK3, HIDHALF), lambda m: (m, 0)),
            pl.BlockSpec((BK3, HIDHALF), lambda m: (m, 0)),
            pl.BlockSpec((BK3, 16), lambda m: (m, 0)),
            pl.BlockSpec((BK3, 16), lambda m: (m, 0)),
            pl.BlockSpec((HIDHALF, SCALAR_DIM), lambda m: (0, 0)),
            pl.BlockSpec((HIDHALF, SCALAR_DIM), lambda m: (0, 0)),
            pl.BlockSpec((1, SCALAR_DIM), lambda m: (0, 0)),
            pl.BlockSpec((SCALAR_DIM, SCALAR_DIM), lambda m: (0, 0)),
        ],
        out_specs=pl.BlockSpec((BK3, SCALAR_DIM), lambda m: (m, 0)),
        out_shape=jax.ShapeDtypeStruct((NT, SCALAR_DIM), f32),
    )(s0a, s1a, s0b, s1b, c0, c1, W3[0:HIDHALF], W3[HIDHALF:HIDDEN], b3r, Wout)

    return out_p[:N_NODES]
